# baseline (speedup 1.0000x reference)
import jax
import jax.numpy as jnp
from jax import lax
from jax.experimental import pallas as pl
from jax.experimental.pallas import tpu as pltpu
from jax.experimental.pallas import tpu_sc as plsc

B = 16384
D = 128
L = 16
NC = 2
NS = 16
NW = NC * NS
BPW = B // NW
C = 128
NCHUNK = BPW // C
NREL = 1000
DW = D // 2


def _sc_body(head_hbm, relidx_hbm, tail_hbm, ent_hbm, relf_hbm, out_hbm,
             idx_h, idx_r, idx_t, hbuf, rbuf, tbuf, out_v, sem_a, sem_b):
    wid = lax.axis_index("s") * NC + lax.axis_index("c")
    base = wid * BPW
    lanes = lax.iota(jnp.int32, L)

    pltpu.async_copy(head_hbm.at[pl.ds(base, BPW)], idx_h, sem_a)
    pltpu.async_copy(relidx_hbm.at[pl.ds(base, BPW)], idx_r, sem_a)
    pltpu.async_copy(tail_hbm.at[pl.ds(base, BPW)], idx_t, sem_a)
    pltpu.make_async_copy(head_hbm.at[pl.ds(base, BPW)], idx_h, sem_a).wait()
    pltpu.make_async_copy(relidx_hbm.at[pl.ds(base, BPW)], idx_r, sem_a).wait()
    pltpu.make_async_copy(tail_hbm.at[pl.ds(base, BPW)], idx_t, sem_a).wait()

    def fire(c, slot, sem):
        pltpu.async_copy(ent_hbm.at[idx_h.at[pl.ds(c * C, C)]],
                         hbuf.at[slot], sem)
        pltpu.async_copy(relf_hbm.at[idx_r.at[pl.ds(c * C, C)]],
                         rbuf.at[slot], sem)
        pltpu.async_copy(ent_hbm.at[idx_t.at[pl.ds(c * C, C)]],
                         tbuf.at[slot], sem)

    def wait(c, slot, sem):
        pltpu.make_async_copy(ent_hbm.at[idx_h.at[pl.ds(c * C, C)]],
                              hbuf.at[slot], sem).wait()
        pltpu.make_async_copy(relf_hbm.at[idx_r.at[pl.ds(c * C, C)]],
                              rbuf.at[slot], sem).wait()
        pltpu.make_async_copy(ent_hbm.at[idx_t.at[pl.ds(c * C, C)]],
                              tbuf.at[slot], sem).wait()

    def compute(c, slot):
        h_ref = hbuf.at[slot]
        r_ref = rbuf.at[slot]
        t_ref = tbuf.at[slot]
        out_base = c * C

        def grp(g, carry2):
            def inner(l, score):
                e = g * L + l
                acc = jnp.zeros((L,), jnp.float32)
                for j in range(D // L):
                    hv = h_ref[e, pl.ds(j * L, L)]
                    rv = r_ref[e, pl.ds(j * L, L)]
                    tv = t_ref[e, pl.ds(j * L, L)]
                    acc = acc + hv * rv * tv
                s = jnp.sum(acc)
                return jnp.where(lanes == l, s, score)

            score = lax.fori_loop(0, L, inner, jnp.zeros((L,), jnp.float32))
            out_v[pl.ds(out_base + g * L, L)] = score
            return carry2

        lax.fori_loop(0, C // L, grp, 0)

    fire(0, 0, sem_a)

    def pair(k, carry):
        c0 = 2 * k
        fire(c0 + 1, 1, sem_b)
        wait(c0, 0, sem_a)
        compute(c0, 0)

        @pl.when(c0 + 2 < NCHUNK)
        def _():
            fire(c0 + 2, 0, sem_a)

        wait(c0 + 1, 1, sem_b)
        compute(c0 + 1, 1)
        return carry

    lax.fori_loop(0, NCHUNK // 2, pair, 0)

    pltpu.sync_copy(out_v, out_hbm.at[pl.ds(base, BPW)])


def kernel(head, relation, tail, entity_embeddings, relation_embeddings):
    mesh = plsc.VectorSubcoreMesh(core_axis_name="c", subcore_axis_name="s",
                                  num_cores=NC, num_subcores=NS)
    kfn = pl.kernel(
        _sc_body,
        out_type=jax.ShapeDtypeStruct((B,), jnp.float32),
        mesh=mesh,
        compiler_params=pltpu.CompilerParams(needs_layout_passes=False),
        scratch_types=[
            pltpu.VMEM((BPW,), jnp.int32),
            pltpu.VMEM((BPW,), jnp.int32),
            pltpu.VMEM((BPW,), jnp.int32),
            pltpu.VMEM((2, C, D), jnp.float32),
            pltpu.VMEM((2, C, D), jnp.float32),
            pltpu.VMEM((2, C, D), jnp.float32),
            pltpu.VMEM((BPW,), jnp.float32),
            pltpu.SemaphoreType.DMA,
            pltpu.SemaphoreType.DMA,
        ],
    )
    return kfn(head, relation, tail, entity_embeddings, relation_embeddings)


# speedup vs baseline: 1.0252x; 1.0252x over previous
import jax
import jax.numpy as jnp
from jax import lax
from jax.experimental import pallas as pl
from jax.experimental.pallas import tpu as pltpu
from jax.experimental.pallas import tpu_sc as plsc

B = 16384
D = 128
L = 16
NC = 2
NS = 16
NW = NC * NS
BPW = B // NW
C = 128
NCHUNK = BPW // C
NREL = 1000
DW = D // 2


def _sc_body(head_hbm, relidx_hbm, tail_hbm, ent_hbm, relf_hbm, out_hbm,
             idx_h, idx_r, idx_t, hbuf, rbuf, tbuf, out_v, sem_a):
    wid = lax.axis_index("s") * NC + lax.axis_index("c")
    base = wid * BPW
    lanes = lax.iota(jnp.int32, L)

    pltpu.async_copy(head_hbm.at[pl.ds(base, BPW)], idx_h, sem_a)
    pltpu.async_copy(relidx_hbm.at[pl.ds(base, BPW)], idx_r, sem_a)
    pltpu.async_copy(tail_hbm.at[pl.ds(base, BPW)], idx_t, sem_a)
    pltpu.make_async_copy(head_hbm.at[pl.ds(base, BPW)], idx_h, sem_a).wait()
    pltpu.make_async_copy(relidx_hbm.at[pl.ds(base, BPW)], idx_r, sem_a).wait()
    pltpu.make_async_copy(tail_hbm.at[pl.ds(base, BPW)], idx_t, sem_a).wait()

    def fire(c, slot):
        pltpu.async_copy(ent_hbm.at[idx_h.at[pl.ds(c * C, C)]],
                         hbuf.at[slot], sem_a)
        pltpu.async_copy(relf_hbm.at[idx_r.at[pl.ds(c * C, C)]],
                         rbuf.at[slot], sem_a)
        pltpu.async_copy(ent_hbm.at[idx_t.at[pl.ds(c * C, C)]],
                         tbuf.at[slot], sem_a)

    fire(0, 0)

    def step(c, carry):
        slot = jnp.bitwise_and(c, 1)
        pltpu.make_async_copy(ent_hbm.at[idx_h.at[pl.ds(c * C, C)]],
                              hbuf.at[slot], sem_a).wait()
        pltpu.make_async_copy(relf_hbm.at[idx_r.at[pl.ds(c * C, C)]],
                              rbuf.at[slot], sem_a).wait()
        pltpu.make_async_copy(ent_hbm.at[idx_t.at[pl.ds(c * C, C)]],
                              tbuf.at[slot], sem_a).wait()

        @pl.when(c + 1 < NCHUNK)
        def _():
            fire(c + 1, jnp.bitwise_and(c + 1, 1))

        h_ref = hbuf.at[slot]
        r_ref = rbuf.at[slot]
        t_ref = tbuf.at[slot]
        out_base = c * C

        def grp(g, carry2):
            def inner(l, score):
                e = g * L + l
                acc = jnp.zeros((L,), jnp.float32)
                for j in range(D // L):
                    hv = h_ref[e, pl.ds(j * L, L)]
                    rv = r_ref[e, pl.ds(j * L, L)]
                    tv = t_ref[e, pl.ds(j * L, L)]
                    acc = acc + hv * rv * tv
                s = jnp.sum(acc)
                return jnp.where(lanes == l, s, score)

            score = lax.fori_loop(0, L, inner, jnp.zeros((L,), jnp.float32))
            out_v[pl.ds(out_base + g * L, L)] = score
            return carry2

        lax.fori_loop(0, C // L, grp, 0)
        return carry

    lax.fori_loop(0, NCHUNK, step, 0)

    pltpu.sync_copy(out_v, out_hbm.at[pl.ds(base, BPW)])


def kernel(head, relation, tail, entity_embeddings, relation_embeddings):
    mesh = plsc.VectorSubcoreMesh(core_axis_name="c", subcore_axis_name="s",
                                  num_cores=NC, num_subcores=NS)
    kfn = pl.kernel(
        _sc_body,
        out_type=jax.ShapeDtypeStruct((B,), jnp.float32),
        mesh=mesh,
        compiler_params=pltpu.CompilerParams(needs_layout_passes=False),
        scratch_types=[
            pltpu.VMEM((BPW,), jnp.int32),
            pltpu.VMEM((BPW,), jnp.int32),
            pltpu.VMEM((BPW,), jnp.int32),
            pltpu.VMEM((2, C, D), jnp.float32),
            pltpu.VMEM((2, C, D), jnp.float32),
            pltpu.VMEM((2, C, D), jnp.float32),
            pltpu.VMEM((BPW,), jnp.float32),
            pltpu.SemaphoreType.DMA,
        ],
    )
    return kfn(head, relation, tail, entity_embeddings, relation_embeddings)
